# baseline (speedup 1.0000x reference)
import functools

import jax
import jax.numpy as jnp
from jax import lax
from jax.experimental import pallas as pl
from jax.experimental.pallas import tpu as pltpu
from jax.experimental.pallas import tpu_sc as plsc

N = 8192
K = 256
M = 8192
BR = 2048
BC = 1024
R = N // BR
C = M // BC
NW = 32
RPW = N // NW


def _cbsq_body(cb_ref, out_ref):
    out_ref[...] = jnp.sum(jnp.square(cb_ref[...]), 0, keepdims=True)


_cbsq_call = pl.pallas_call(
    _cbsq_body,
    grid=(C,),
    in_specs=[pl.BlockSpec((K, BC), lambda c: (0, c))],
    out_specs=pl.BlockSpec((1, BC), lambda c: (0, c)),
    out_shape=jax.ShapeDtypeStruct((1, M), jnp.float32),
)


def _argmin_body(x_ref, cb_ref, xsq_ref, cbsq_ref, out_ref,
                 best_val, best_idx, gidx_ref):
    c = pl.program_id(1)
    nm = jnp.dot(x_ref[...], cb_ref[...], preferred_element_type=jnp.float32)
    xsq = xsq_ref[...]
    cbsq = cbsq_ref[...]

    G = BC // 128
    def dgrp(g):
        lo, hi = g * 128, (g + 1) * 128
        return (xsq + nm[:, lo:hi]) + cbsq[:, lo:hi]
    d0 = dgrp(0)
    d1 = dgrp(1)
    runidx = jnp.where(d1 < d0, 1, 0)
    runmin = jnp.minimum(d1, d0)
    for g in range(2, G):
        dg = dgrp(g)
        lt = dg < runmin
        runmin = jnp.where(lt, dg, runmin)
        runidx = jnp.where(lt, g, runidx)
    dmin = jnp.min(runmin, axis=1, keepdims=True)
    lanes = lax.broadcasted_iota(jnp.int32, (BR, 128), 1)
    cand = jnp.where(runmin == dmin, runidx * 128 + lanes, BC)
    gidx_ref[...] = jnp.min(cand, axis=1, keepdims=True) + c * BC

    @pl.when(c == 0)
    def _():
        best_val[...] = dmin
        best_idx[...] = gidx_ref[...]

    @pl.when(c > 0)
    def _():
        upd = dmin < best_val[...]
        best_val[...] = jnp.where(upd, dmin, best_val[...])
        best_idx[...] = jnp.where(upd, gidx_ref[...], best_idx[...])

    @pl.when(c == C - 1)
    def _():
        out_ref[...] = best_idx[...]


_argmin_call = pl.pallas_call(
    _argmin_body,
    grid=(R, C),
    in_specs=[
        pl.BlockSpec((BR, K), lambda r, c: (r, 0)),
        pl.BlockSpec((K, BC), lambda r, c: (0, c)),
        pl.BlockSpec((BR, 1), lambda r, c: (r, 0)),
        pl.BlockSpec((1, BC), lambda r, c: (0, c)),
    ],
    out_specs=pl.BlockSpec((BR, 1), lambda r, c: (r, 0)),
    out_shape=jax.ShapeDtypeStruct((N, 1), jnp.int32),
    scratch_shapes=[
        pltpu.VMEM((BR, 1), jnp.float32),
        pltpu.VMEM((BR, 1), jnp.int32),
        pltpu.VMEM((BR, 1), jnp.int32),
    ],
)


def _sc_gather_hist_body(ct_hbm, idx_hbm, q_hbm, counts_hbm,
                         idx_v, rows_v, ones_v, zeros_v, shared_counts, sem):
    core = lax.axis_index("c")
    sub = lax.axis_index("s")
    wid = sub * 2 + core
    base = wid * RPW

    for j in range(RPW // 16):
        ones_v[pl.ds(j * 16, 16)] = jnp.full((16,), 1.0, jnp.float32)
    for j in range(512 // 16):
        zeros_v[pl.ds(j * 16, 16)] = jnp.zeros((16,), jnp.float32)

    pltpu.sync_copy(idx_hbm.at[pl.ds(base, RPW)], idx_v)
    pltpu.async_copy(ct_hbm.at[idx_v], rows_v, sem).wait()
    pltpu.sync_copy(rows_v, q_hbm.at[pl.ds(base, RPW)])

    pltpu.sync_copy(zeros_v, shared_counts.at[pl.ds(sub * 512, 512)])
    plsc.subcore_barrier()
    pltpu.sync_copy(ones_v, shared_counts.at[idx_v], add=True)
    plsc.subcore_barrier()

    @pl.when(sub == 0)
    def _():
        pltpu.sync_copy(shared_counts, counts_hbm.at[core])


@functools.cache
def _sc_gather_hist():
    return pl.kernel(
        _sc_gather_hist_body,
        out_type=[
            jax.ShapeDtypeStruct((N, K), jnp.float32),
            jax.ShapeDtypeStruct((2, M), jnp.float32),
        ],
        mesh=plsc.VectorSubcoreMesh(core_axis_name="c", subcore_axis_name="s"),
        scratch_types=[
            pltpu.VMEM((RPW,), jnp.int32),
            pltpu.VMEM((RPW, K), jnp.float32),
            pltpu.VMEM((RPW,), jnp.float32),
            pltpu.VMEM((512,), jnp.float32),
            pltpu.VMEM_SHARED((M,), jnp.float32),
            pltpu.SemaphoreType.DMA,
        ],
    )

FB = N // 8


def _finalize_body(x_ref, q_ref, counts_ref, ste_ref, loss_ref, perp_ref, acc):
    i = pl.program_id(0)
    x = x_ref[...]
    diff = q_ref[...] - x
    ste_ref[...] = x + diff
    part = jnp.sum(diff * diff)

    @pl.when(i == 0)
    def _():
        acc[0, 0] = part

    @pl.when(i > 0)
    def _():
        acc[0, 0] = acc[0, 0] + part

    @pl.when(i == 7)
    def _():
        loss_ref[...] = jnp.reshape(acc[0, 0] / 2097152.0, (1, 1))
        counts = counts_ref[0:1, :] + counts_ref[1:2, :]
        avg = counts * (1.0 / 8192.0)
        ent = jnp.sum(avg * jnp.log(avg + 1e-10))
        perp_ref[...] = jnp.reshape(jnp.exp(-ent), (1, 1))


_finalize_call = pl.pallas_call(
    _finalize_body,
    grid=(8,),
    in_specs=[
        pl.BlockSpec((FB, K), lambda i: (i, 0)),
        pl.BlockSpec((FB, K), lambda i: (i, 0)),
        pl.BlockSpec((2, M), lambda i: (0, 0)),
    ],
    out_specs=[
        pl.BlockSpec((FB, K), lambda i: (i, 0)),
        pl.BlockSpec((1, 1), lambda i: (0, 0)),
        pl.BlockSpec((1, 1), lambda i: (0, 0)),
    ],
    out_shape=[
        jax.ShapeDtypeStruct((N, K), jnp.float32),
        jax.ShapeDtypeStruct((1, 1), jnp.float32),
        jax.ShapeDtypeStruct((1, 1), jnp.float32),
    ],
    scratch_shapes=[pltpu.SMEM((1, 1), jnp.float32)],
)


def kernel(inputs, codebook):
    x = inputs.reshape(N, K)
    xsq = jnp.sum(jnp.square(inputs), 2).reshape(N, 1)
    cbsq = _cbsq_call(codebook)
    xn = (x * -2.0).astype(jnp.bfloat16)
    cbb = codebook.astype(jnp.bfloat16)
    idx = _argmin_call(xn, cbb, xsq, cbsq).reshape(N)
    ct = codebook.T
    q, counts = _sc_gather_hist()(ct, idx)
    ste, loss, perp = _finalize_call(x, q, counts)
    loss0 = loss[0, 0]
    return (ste.reshape(inputs.shape), perp[0, 0], loss0, 0.25 * loss0)


# speedup vs baseline: 1.0622x; 1.0622x over previous
import functools

import jax
import jax.numpy as jnp
from jax import lax
from jax.experimental import pallas as pl
from jax.experimental.pallas import tpu as pltpu
from jax.experimental.pallas import tpu_sc as plsc

N = 8192
K = 256
M = 8192
BR = 2048
BC = 1024
R = N // BR
C = M // BC
NW = 32
RPW = N // NW


def _argmin_body(x_ref, cb_ref, xsq_ref, out_ref,
                 best_val, best_idx, gidx_ref):
    c = pl.program_id(1)
    cb = cb_ref[...]
    cbsq = jnp.sum(jnp.square(cb), 0, keepdims=True)

    xb = (x_ref[...] * -2.0).astype(jnp.bfloat16)
    cbb = cb.astype(jnp.bfloat16)
    nm = jnp.dot(xb, cbb, preferred_element_type=jnp.float32)
    xsq = xsq_ref[...]

    G = BC // 128
    def dgrp(g):
        lo, hi = g * 128, (g + 1) * 128
        return (xsq + nm[:, lo:hi]) + cbsq[:, lo:hi]
    d0 = dgrp(0)
    d1 = dgrp(1)
    runidx = jnp.where(d1 < d0, 1, 0)
    runmin = jnp.minimum(d1, d0)
    for g in range(2, G):
        dg = dgrp(g)
        lt = dg < runmin
        runmin = jnp.where(lt, dg, runmin)
        runidx = jnp.where(lt, g, runidx)
    dmin = jnp.min(runmin, axis=1, keepdims=True)
    lanes = lax.broadcasted_iota(jnp.int32, (BR, 128), 1)
    cand = jnp.where(runmin == dmin, runidx * 128 + lanes, BC)
    gidx_ref[...] = jnp.min(cand, axis=1, keepdims=True) + c * BC

    @pl.when(c == 0)
    def _():
        best_val[...] = dmin
        best_idx[...] = gidx_ref[...]

    @pl.when(c > 0)
    def _():
        upd = dmin < best_val[...]
        best_val[...] = jnp.where(upd, dmin, best_val[...])
        best_idx[...] = jnp.where(upd, gidx_ref[...], best_idx[...])

    @pl.when(c == C - 1)
    def _():
        out_ref[...] = best_idx[...]


_argmin_call = pl.pallas_call(
    _argmin_body,
    grid=(R, C),
    in_specs=[
        pl.BlockSpec((BR, K), lambda r, c: (r, 0)),
        pl.BlockSpec((K, BC), lambda r, c: (0, c)),
        pl.BlockSpec((BR, 1), lambda r, c: (r, 0)),
    ],
    out_specs=pl.BlockSpec((BR, 1), lambda r, c: (r, 0)),
    out_shape=jax.ShapeDtypeStruct((N, 1), jnp.int32),
    scratch_shapes=[
        pltpu.VMEM((BR, 1), jnp.float32),
        pltpu.VMEM((BR, 1), jnp.int32),
        pltpu.VMEM((BR, 1), jnp.int32),
    ],
)


def _sc_gather_hist_body(ct_hbm, idx_hbm, q_hbm, counts_hbm,
                         idx_v, rows_v, ones_v, zeros_v, shared_counts, sem):
    core = lax.axis_index("c")
    sub = lax.axis_index("s")
    wid = sub * 2 + core
    base = wid * RPW

    for j in range(RPW // 16):
        ones_v[pl.ds(j * 16, 16)] = jnp.full((16,), 1.0, jnp.float32)
    for j in range(512 // 16):
        zeros_v[pl.ds(j * 16, 16)] = jnp.zeros((16,), jnp.float32)

    pltpu.sync_copy(idx_hbm.at[pl.ds(base, RPW)], idx_v)
    pltpu.async_copy(ct_hbm.at[idx_v], rows_v, sem).wait()
    pltpu.sync_copy(rows_v, q_hbm.at[pl.ds(base, RPW)])

    pltpu.sync_copy(zeros_v, shared_counts.at[pl.ds(sub * 512, 512)])
    plsc.subcore_barrier()
    pltpu.sync_copy(ones_v, shared_counts.at[idx_v], add=True)
    plsc.subcore_barrier()

    @pl.when(sub == 0)
    def _():
        pltpu.sync_copy(shared_counts, counts_hbm.at[core])


@functools.cache
def _sc_gather_hist():
    return pl.kernel(
        _sc_gather_hist_body,
        out_type=[
            jax.ShapeDtypeStruct((N, K), jnp.float32),
            jax.ShapeDtypeStruct((2, M), jnp.float32),
        ],
        mesh=plsc.VectorSubcoreMesh(core_axis_name="c", subcore_axis_name="s"),
        scratch_types=[
            pltpu.VMEM((RPW,), jnp.int32),
            pltpu.VMEM((RPW, K), jnp.float32),
            pltpu.VMEM((RPW,), jnp.float32),
            pltpu.VMEM((512,), jnp.float32),
            pltpu.VMEM_SHARED((M,), jnp.float32),
            pltpu.SemaphoreType.DMA,
        ],
    )

FB = N // 8


def _finalize_body(x_ref, q_ref, counts_ref, ste_ref, loss_ref, perp_ref, acc):
    i = pl.program_id(0)
    x = x_ref[...]
    diff = q_ref[...] - x
    ste_ref[...] = x + diff
    part = jnp.sum(diff * diff)

    @pl.when(i == 0)
    def _():
        acc[0, 0] = part

    @pl.when(i > 0)
    def _():
        acc[0, 0] = acc[0, 0] + part

    @pl.when(i == 7)
    def _():
        loss_ref[...] = jnp.reshape(acc[0, 0] / 2097152.0, (1, 1))
        counts = counts_ref[0:1, :] + counts_ref[1:2, :]
        avg = counts * (1.0 / 8192.0)
        ent = jnp.sum(avg * jnp.log(avg + 1e-10))
        perp_ref[...] = jnp.reshape(jnp.exp(-ent), (1, 1))


_finalize_call = pl.pallas_call(
    _finalize_body,
    grid=(8,),
    in_specs=[
        pl.BlockSpec((FB, K), lambda i: (i, 0)),
        pl.BlockSpec((FB, K), lambda i: (i, 0)),
        pl.BlockSpec((2, M), lambda i: (0, 0)),
    ],
    out_specs=[
        pl.BlockSpec((FB, K), lambda i: (i, 0)),
        pl.BlockSpec((1, 1), lambda i: (0, 0)),
        pl.BlockSpec((1, 1), lambda i: (0, 0)),
    ],
    out_shape=[
        jax.ShapeDtypeStruct((N, K), jnp.float32),
        jax.ShapeDtypeStruct((1, 1), jnp.float32),
        jax.ShapeDtypeStruct((1, 1), jnp.float32),
    ],
    scratch_shapes=[pltpu.SMEM((1, 1), jnp.float32)],
)


def kernel(inputs, codebook):
    x = inputs.reshape(N, K)
    xsq = jnp.sum(jnp.square(inputs), 2).reshape(N, 1)
    idx = _argmin_call(x, codebook, xsq).reshape(N)
    ct = codebook.T
    q, counts = _sc_gather_hist()(ct, idx)
    ste, loss, perp = _finalize_call(x, q, counts)
    loss0 = loss[0, 0]
    return (ste.reshape(inputs.shape), perp[0, 0], loss0, 0.25 * loss0)
